# baseline (speedup 1.0000x reference)
import jax
import jax.numpy as jnp
from jax import lax
from jax.experimental import pallas as pl
from jax.experimental.pallas import tpu as pltpu
from jax.experimental.pallas import tpu_sc as plsc

NC = 2
NS = 16
L = 16
D_MODEL = 1024
CHUNK = 32


def _sc_embed_add(n_tokens):
    nw = NC * NS
    tok_per_w = n_tokens // nw
    n_chunks = tok_per_w // CHUNK
    mesh = plsc.VectorSubcoreMesh(core_axis_name="c", subcore_axis_name="s")

    def body(x_hbm, idx_hbm, table_hbm, out_hbm, table_v, idx_v, buf):
        wid = lax.axis_index("s") * NC + lax.axis_index("c")
        base = wid * tok_per_w
        pltpu.sync_copy(table_hbm, table_v)
        pltpu.sync_copy(idx_hbm.at[pl.ds(base, tok_per_w)], idx_v)

        def chunk_body(ci, _):
            tok0 = base + ci * CHUNK
            pltpu.sync_copy(x_hbm.at[pl.ds(tok0, CHUNK)], buf)

            def grp_body(g, _):
                iv = idx_v[pl.ds(ci * CHUNK + g * L, L)]
                for t in range(L):
                    s = iv[t]

                    def col_body(j, _, s=s, t=t, g=g):
                        e = table_v[s, pl.ds(j * L, L)]
                        plsc.addupdate(buf.at[g * L + t, pl.ds(j * L, L)], e)
                        return 0

                    lax.fori_loop(0, D_MODEL // L, col_body, 0)
                return 0

            lax.fori_loop(0, CHUNK // L, grp_body, 0)
            pltpu.sync_copy(buf, out_hbm.at[pl.ds(tok0, CHUNK)])
            return 0

        lax.fori_loop(0, n_chunks, chunk_body, 0)

    return pl.kernel(
        body,
        out_type=jax.ShapeDtypeStruct((n_tokens, D_MODEL), jnp.float32),
        mesh=mesh,
        scratch_types=[
            pltpu.VMEM((4, D_MODEL), jnp.float32),
            pltpu.VMEM((tok_per_w,), jnp.int32),
            pltpu.VMEM((CHUNK, D_MODEL), jnp.float32),
        ],
    )


def kernel(x, lirads, table):
    b, s, d = x.shape
    n = b * s
    xf = x.reshape(n, d)
    idx = lirads.reshape(n).astype(jnp.int32)
    out = _sc_embed_add(n)(xf, idx, table)
    return out.reshape(b, s, d)
